# Initial kernel scaffold; baseline (speedup 1.0000x reference)
import jax
import jax.numpy as jnp
from jax.experimental import pallas as pl


def kernel(nodes, table):
    raise NotImplementedError("write your pallas kernel here")


# speedup vs baseline: 1.7961x; 1.7961x over previous
import functools

import jax
import jax.numpy as jnp
from jax import lax
from jax.experimental import pallas as pl
from jax.experimental.pallas import tpu as pltpu
from jax.experimental.pallas import tpu_sc as plsc

NUM_NODES = 1_000_000
EMBED_DIM = 64
BATCH = 16384
HIST = 50
TOTAL = BATCH * HIST

_info = plsc.get_sparse_core_info()
NC, NS = _info.num_cores, _info.num_subcores
NW = NC * NS
B_PER_W = TOTAL // NW
CHUNK = 512
N_CHUNKS = B_PER_W // CHUNK


@functools.partial(jax.jit, static_argnames=())
def _gather(table, idx):
    mesh = plsc.VectorSubcoreMesh(core_axis_name="c", subcore_axis_name="s")

    @functools.partial(
        pl.kernel,
        mesh=mesh,
        out_type=jax.ShapeDtypeStruct((TOTAL, EMBED_DIM), jnp.float32),
        scratch_types=[
            pltpu.VMEM((CHUNK,), jnp.int32),
            pltpu.VMEM((CHUNK, EMBED_DIM), jnp.float32),
            pltpu.SemaphoreType.DMA,
        ],
        compiler_params=pltpu.CompilerParams(use_tc_tiling_on_sc=False),
    )
    def body(table_hbm, idx_hbm, out_hbm, idx_v, rows_v, sem):
        wid = lax.axis_index("s") * NC + lax.axis_index("c")
        base = wid * B_PER_W

        def chunk_step(i, carry):
            off = base + i * CHUNK
            pltpu.sync_copy(idx_hbm.at[pl.ds(off, CHUNK)], idx_v)
            pltpu.async_copy(table_hbm.at[idx_v], rows_v, sem).wait()
            pltpu.sync_copy(rows_v, out_hbm.at[pl.ds(off, CHUNK)])
            return carry

        lax.fori_loop(0, N_CHUNKS, chunk_step, 0)

    return body(table, idx)


def kernel(nodes, table):
    idx = nodes.reshape(-1).astype(jnp.int32)
    out = _gather(table, idx)
    return out.reshape(BATCH, HIST, EMBED_DIM)


# speedup vs baseline: 1.8769x; 1.0450x over previous
import functools

import jax
import jax.numpy as jnp
from jax import lax
from jax.experimental import pallas as pl
from jax.experimental.pallas import tpu as pltpu
from jax.experimental.pallas import tpu_sc as plsc

NUM_NODES = 1_000_000
EMBED_DIM = 64
BATCH = 16384
HIST = 50
TOTAL = BATCH * HIST

_info = plsc.get_sparse_core_info()
NC, NS = _info.num_cores, _info.num_subcores
NW = NC * NS
B_PER_W = TOTAL // NW
CHUNK = 800
N_CHUNKS = B_PER_W // CHUNK


def _gather(table, idx):
    mesh = plsc.VectorSubcoreMesh(core_axis_name="c", subcore_axis_name="s")

    @functools.partial(
        pl.kernel,
        mesh=mesh,
        out_type=jax.ShapeDtypeStruct((TOTAL, EMBED_DIM), jnp.float32),
        scratch_types=[
            pltpu.VMEM((CHUNK,), jnp.int32),
            pltpu.VMEM((CHUNK,), jnp.int32),
            pltpu.VMEM((CHUNK, EMBED_DIM), jnp.float32),
            pltpu.VMEM((CHUNK, EMBED_DIM), jnp.float32),
            pltpu.SemaphoreType.DMA,
            pltpu.SemaphoreType.DMA,
            pltpu.SemaphoreType.DMA,
            pltpu.SemaphoreType.DMA,
        ],
        compiler_params=pltpu.CompilerParams(use_tc_tiling_on_sc=False),
    )
    def body(table_hbm, idx_hbm, out_hbm, idx_v0, idx_v1, rows_v0, rows_v1,
             sem_g0, sem_g1, sem_o0, sem_o1):
        wid = lax.axis_index("s") * NC + lax.axis_index("c")
        base = wid * B_PER_W
        idx_v = (idx_v0, idx_v1)
        rows_v = (rows_v0, rows_v1)
        sem_g = (sem_g0, sem_g1)
        sem_o = (sem_o0, sem_o1)

        def idx_copy(i, b):
            pltpu.sync_copy(idx_hbm.at[pl.ds(base + i * CHUNK, CHUNK)],
                            idx_v[b])

        def gather_start(b):
            pltpu.async_copy(table_hbm.at[idx_v[b]], rows_v[b], sem_g[b])

        def gather_wait(b):
            pltpu.make_async_copy(table_hbm.at[idx_v[b]], rows_v[b],
                                  sem_g[b]).wait()

        def out_start(i, b):
            pltpu.async_copy(rows_v[b],
                             out_hbm.at[pl.ds(base + i * CHUNK, CHUNK)],
                             sem_o[b])

        def out_wait(i, b):
            pltpu.make_async_copy(rows_v[b],
                                  out_hbm.at[pl.ds(base + i * CHUNK, CHUNK)],
                                  sem_o[b]).wait()

        idx_copy(0, 0)
        gather_start(0)
        idx_copy(1, 1)
        gather_start(1)

        def pair_step(j, carry):
            for b in range(2):
                i = 2 * j + b
                gather_wait(b)
                out_start(i, b)
                idx_copy(i + 2, b)
                out_wait(i, b)
                gather_start(b)
            return carry

        lax.fori_loop(0, N_CHUNKS // 2 - 1, pair_step, 0)

        for b in range(2):
            i = N_CHUNKS - 2 + b
            gather_wait(b)
            out_start(i, b)
            out_wait(i, b)

    return body(table, idx)


def kernel(nodes, table):
    idx = nodes.reshape(-1).astype(jnp.int32)
    out = _gather(table, idx)
    return out.reshape(BATCH, HIST, EMBED_DIM)
